# baseline (speedup 1.0000x reference)
import functools

import jax
import jax.numpy as jnp
from jax import lax
from jax.experimental import pallas as pl
from jax.experimental.pallas import tpu as pltpu
from jax.experimental.pallas import tpu_sc as plsc

N = 10000
E = 320000
H = 128
DE = 4
L = 3
DLAT = 64
G = 64

NW = 32
PW = E // NW
EBLK = 128
NFULL = PW // EBLK
TAIL = PW - NFULL * EBLK
EMAIN = NW * NFULL * EBLK
RPS = 624

CH = (30, 48)
E0 = NW * CH[0] * EBLK
E1M = NW * CH[1] * EBLK
E1P = E1M + 768

BE = 2000
BN = 2000

def _ln(x, g, b, eps=1e-05):
    mu = jnp.mean(x, axis=-1, keepdims=True)
    var = jnp.mean((x - mu) ** 2, axis=-1, keepdims=True)
    return (x - mu) / jnp.sqrt(var + eps) * g + b


def _silu(x):
    return x * jax.nn.sigmoid(x)


def _pack_bf16_pair(h):
    ua = lax.bitcast_convert_type(h[:, :H // 2], jnp.uint32)
    ub = lax.bitcast_convert_type(h[:, H // 2:], jnp.uint32)
    ua = (ua + jnp.uint32(0x8000)) >> 16
    ub = (ub + jnp.uint32(0x8000)) >> 16
    return lax.bitcast_convert_type(ua | (ub << 16), jnp.int32)


def _unpack_bf16_pair(w):
    u = lax.bitcast_convert_type(w, jnp.uint32)
    lo = lax.bitcast_convert_type(u << 16, jnp.float32)
    hi = lax.bitcast_convert_type(u & jnp.uint32(0xFFFF0000), jnp.float32)
    return lo.astype(jnp.bfloat16), hi.astype(jnp.bfloat16)


@functools.lru_cache(maxsize=None)
def _sc_gather_kernel(chunk):
    mesh = plsc.VectorSubcoreMesh(core_axis_name="c", subcore_axis_name="s")
    nb = NFULL if chunk == -1 else CH[chunk]
    eout = E if chunk == -1 else (E0 if chunk == 0 else E1P)
    tout0 = EMAIN if chunk == -1 else E1M

    @functools.partial(
        pl.kernel,
        out_type=(
            jax.ShapeDtypeStruct((eout, H), jnp.float32),
            jax.ShapeDtypeStruct((eout, H), jnp.float32),
        ),
        mesh=mesh,
        scratch_types=[
            pltpu.VMEM((nb, EBLK), jnp.int32),
            pltpu.VMEM((nb, EBLK), jnp.int32),
            pltpu.VMEM((EBLK, H), jnp.float32),
            pltpu.VMEM((EBLK, H), jnp.float32),
            pltpu.VMEM((EBLK, H), jnp.float32),
            pltpu.VMEM((EBLK, H), jnp.float32),
            pltpu.VMEM((TAIL,), jnp.int32),
            pltpu.VMEM((TAIL,), jnp.int32),
            pltpu.VMEM((TAIL, H), jnp.float32),
            pltpu.VMEM((TAIL, H), jnp.float32),
        ] + [pltpu.SemaphoreType.DMA] * 8,
    )
    def k(h_hbm, src3_hbm, dst3_hbm, src_hbm, dst_hbm, os_hbm, od_hbm,
          is2, id2, bs0, bd0, bs1, bd1, si_t, di_t, sr_t, dr_t,
          gs0, gd0, gs1, gd1, ws0, wd0, ws1, wd1):
        wid = lax.axis_index("c") * 16 + lax.axis_index("s")
        base = wid * (nb * EBLK)
        pltpu.sync_copy(src3_hbm.at[wid], is2)
        pltpu.sync_copy(dst3_hbm.at[wid], id2)

        @pl.loop(0, nb // 2)
        def _(p):
            b0 = 2 * p
            b1 = b0 + 1
            o0 = base + b0 * EBLK
            o1 = base + b1 * EBLK

            @pl.when(p > 0)
            def _():
                pltpu.make_async_copy(bs0, os_hbm.at[pl.ds(o0 - 2 * EBLK, EBLK)],
                                      ws0).wait()
                pltpu.make_async_copy(bd0, od_hbm.at[pl.ds(o0 - 2 * EBLK, EBLK)],
                                      wd0).wait()

            g0 = pltpu.async_copy(h_hbm.at[is2.at[b0]], bs0, gs0)
            g1 = pltpu.async_copy(h_hbm.at[id2.at[b0]], bd0, gd0)

            @pl.when(p > 0)
            def _():
                pltpu.make_async_copy(bs1, os_hbm.at[pl.ds(o1 - 2 * EBLK, EBLK)],
                                      ws1).wait()
                pltpu.make_async_copy(bd1, od_hbm.at[pl.ds(o1 - 2 * EBLK, EBLK)],
                                      wd1).wait()

            g2 = pltpu.async_copy(h_hbm.at[is2.at[b1]], bs1, gs1)
            g3 = pltpu.async_copy(h_hbm.at[id2.at[b1]], bd1, gd1)
            g0.wait()
            g1.wait()
            pltpu.async_copy(bs0, os_hbm.at[pl.ds(o0, EBLK)], ws0)
            pltpu.async_copy(bd0, od_hbm.at[pl.ds(o0, EBLK)], wd0)
            g2.wait()
            g3.wait()
            pltpu.async_copy(bs1, os_hbm.at[pl.ds(o1, EBLK)], ws1)
            pltpu.async_copy(bd1, od_hbm.at[pl.ds(o1, EBLK)], wd1)

        lo0 = base + (nb - 2) * EBLK
        lo1 = base + (nb - 1) * EBLK
        pltpu.make_async_copy(bs0, os_hbm.at[pl.ds(lo0, EBLK)], ws0).wait()
        pltpu.make_async_copy(bd0, od_hbm.at[pl.ds(lo0, EBLK)], wd0).wait()
        pltpu.make_async_copy(bs1, os_hbm.at[pl.ds(lo1, EBLK)], ws1).wait()
        pltpu.make_async_copy(bd1, od_hbm.at[pl.ds(lo1, EBLK)], wd1).wait()

        if chunk != 0:
            tin = EMAIN + wid * TAIL
            tout = tout0 + wid * TAIL
            pltpu.sync_copy(src_hbm.at[pl.ds(tin, TAIL)], si_t)
            pltpu.sync_copy(dst_hbm.at[pl.ds(tin, TAIL)], di_t)
            pltpu.sync_copy(h_hbm.at[si_t], sr_t)
            pltpu.sync_copy(h_hbm.at[di_t], dr_t)
            pltpu.sync_copy(sr_t, os_hbm.at[pl.ds(tout, TAIL)])
            pltpu.sync_copy(dr_t, od_hbm.at[pl.ds(tout, TAIL)])

    return k


def _sc_gather(chunk, h, src3, dst3, src, dst):
    return _sc_gather_kernel(chunk)(h, src3, dst3, src, dst)


@functools.lru_cache(maxsize=None)
def _sc_scatter_kernel(chunk):
    mesh = plsc.VectorSubcoreMesh(core_axis_name="c", subcore_axis_name="s")
    nb = NFULL if chunk == -1 else CH[chunk]
    ein = E if chunk == -1 else (E0 if chunk == 0 else E1P)
    tm0 = EMAIN if chunk == -1 else E1M

    @functools.partial(
        pl.kernel,
        out_type=jax.ShapeDtypeStruct((2, N, H), jnp.float32),
        mesh=mesh,
        scratch_types=[
            pltpu.VMEM((nb, EBLK), jnp.int32),
            pltpu.VMEM((EBLK, H), jnp.float32),
            pltpu.VMEM((EBLK, H), jnp.float32),
            pltpu.VMEM((TAIL,), jnp.int32),
            pltpu.VMEM((TAIL, H), jnp.float32),
            pltpu.VMEM_SHARED((N, H), jnp.float32),
        ] + [pltpu.SemaphoreType.DMA] * 2,
    )
    def k(m_hbm, dst3_hbm, dst_hbm, z_hbm, out_hbm,
          id2, ba, bb, di_t, mr_t, acc_sh, sa, sb):
        cid = lax.axis_index("c")
        sid = lax.axis_index("s")
        wid = cid * 16 + sid
        r0 = sid * RPS

        @pl.when(sid < 15)
        def _():
            pltpu.sync_copy(z_hbm.at[pl.ds(r0, RPS)], acc_sh.at[pl.ds(r0, RPS)])

        @pl.when(sid == 15)
        def _():
            pltpu.sync_copy(z_hbm.at[pl.ds(15 * RPS, N - 15 * RPS)],
                            acc_sh.at[pl.ds(15 * RPS, N - 15 * RPS)])

        pltpu.sync_copy(dst3_hbm.at[wid], id2)
        plsc.subcore_barrier()

        base = wid * (nb * EBLK)
        pltpu.async_copy(m_hbm.at[pl.ds(base, EBLK)], ba, sa).wait()

        @pl.loop(0, nb // 2)
        def _(p):
            b0 = 2 * p
            o1 = base + (b0 + 1) * EBLK
            lb = pltpu.async_copy(m_hbm.at[pl.ds(o1, EBLK)], bb, sb)
            pltpu.sync_copy(ba, acc_sh.at[id2.at[b0]], add=True)
            lb.wait()

            @pl.when(p < nb // 2 - 1)
            def _():
                o2 = base + (b0 + 2) * EBLK
                la = pltpu.async_copy(m_hbm.at[pl.ds(o2, EBLK)], ba, sa)
                pltpu.sync_copy(bb, acc_sh.at[id2.at[b0 + 1]], add=True)
                la.wait()

            @pl.when(p == nb // 2 - 1)
            def _():
                pltpu.sync_copy(bb, acc_sh.at[id2.at[b0 + 1]], add=True)

        if chunk != 0:
            tin = EMAIN + wid * TAIL
            tm = tm0 + wid * TAIL
            pltpu.sync_copy(dst_hbm.at[pl.ds(tin, TAIL)], di_t)
            pltpu.sync_copy(m_hbm.at[pl.ds(tm, TAIL)], mr_t)
            pltpu.sync_copy(mr_t, acc_sh.at[di_t], add=True)

        plsc.subcore_barrier()

        @pl.when(sid < 15)
        def _():
            pltpu.sync_copy(acc_sh.at[pl.ds(r0, RPS)],
                            out_hbm.at[cid, pl.ds(r0, RPS)])

        @pl.when(sid == 15)
        def _():
            pltpu.sync_copy(acc_sh.at[pl.ds(15 * RPS, N - 15 * RPS)],
                            out_hbm.at[cid, pl.ds(15 * RPS, N - 15 * RPS)])

    return k


def _sc_scatter(chunk, m, dst3, dst, zeros_nh):
    return _sc_scatter_kernel(chunk)(m, dst3, dst, zeros_nh)


def _embed(xp, Wp, b, g, be):
    def body(x_ref, W_ref, b_ref, g_ref, be_ref, o_ref):
        h = jnp.dot(x_ref[...], W_ref[...], preferred_element_type=jnp.float32)
        h = h + b_ref[...]
        o_ref[...] = _silu(_ln(h, g_ref[...], be_ref[...]))

    return pl.pallas_call(
        body,
        grid=(N // BN,),
        in_specs=[
            pl.BlockSpec((BN, 8), lambda i: (i, 0)),
            pl.BlockSpec((8, H), lambda i: (0, 0)),
            pl.BlockSpec((1, H), lambda i: (0, 0)),
            pl.BlockSpec((1, H), lambda i: (0, 0)),
            pl.BlockSpec((1, H), lambda i: (0, 0)),
        ],
        out_specs=pl.BlockSpec((BN, H), lambda i: (i, 0)),
        out_shape=jax.ShapeDtypeStruct((N, H), jnp.float32),
    )(xp, Wp, b, g, be)


def _edge_mlp(hd, hs, ea, W1d, W1s, W1e, b1, W2, b2):
    def body(hd_ref, hs_ref, ea_ref, W1d_ref, W1s_ref, W1e_ref, b1_ref,
             W2_ref, b2_ref, o_ref):
        bf16 = jnp.bfloat16
        z = jnp.dot(hd_ref[...].astype(bf16), W1d_ref[...],
                    preferred_element_type=jnp.float32)
        z = z + jnp.dot(hs_ref[...].astype(bf16), W1s_ref[...],
                        preferred_element_type=jnp.float32)
        z = z + jnp.dot(ea_ref[...], W1e_ref[...],
                        preferred_element_type=jnp.float32)
        z = _silu(z + b1_ref[...]).astype(bf16)
        o_ref[...] = jnp.dot(z, W2_ref[...], preferred_element_type=jnp.float32) + b2_ref[...]

    ne = hd.shape[0]
    return pl.pallas_call(
        body,
        grid=(ne // BE,),
        in_specs=[
            pl.BlockSpec((BE, H), lambda i: (i, 0)),
            pl.BlockSpec((BE, H), lambda i: (i, 0)),
            pl.BlockSpec((BE, 8), lambda i: (i, 0)),
            pl.BlockSpec((H, 2 * H), lambda i: (0, 0)),
            pl.BlockSpec((H, 2 * H), lambda i: (0, 0)),
            pl.BlockSpec((8, 2 * H), lambda i: (0, 0)),
            pl.BlockSpec((1, 2 * H), lambda i: (0, 0)),
            pl.BlockSpec((2 * H, H), lambda i: (0, 0)),
            pl.BlockSpec((1, H), lambda i: (0, 0)),
        ],
        out_specs=pl.BlockSpec((BE, H), lambda i: (i, 0)),
        out_shape=jax.ShapeDtypeStruct((ne, H), jnp.float32),
    )(hd, hs, ea, W1d, W1s, W1e, b1, W2, b2)


def _node_update(h, a0, a1, Wgh, Wga, bg, g, b):
    def body(h_ref, a0_ref, a1_ref, Wgh_ref, Wga_ref, bg_ref,
             g_ref, b_ref, o_ref):
        h = h_ref[...]
        a = a0_ref[...] + a1_ref[...]
        gate = jnp.dot(h, Wgh_ref[...], preferred_element_type=jnp.float32)
        gate = gate + jnp.dot(a, Wga_ref[...], preferred_element_type=jnp.float32)
        gate = jax.nn.sigmoid(gate + bg_ref[...])
        h_new = gate * a + (1.0 - gate) * h
        o_ref[...] = _ln(h + h_new, g_ref[...], b_ref[...])

    return pl.pallas_call(
        body,
        grid=(N // BN,),
        in_specs=[
            pl.BlockSpec((BN, H), lambda i: (i, 0)),
            pl.BlockSpec((BN, H), lambda i: (i, 0)),
            pl.BlockSpec((BN, H), lambda i: (i, 0)),
            pl.BlockSpec((H, H), lambda i: (0, 0)),
            pl.BlockSpec((H, H), lambda i: (0, 0)),
            pl.BlockSpec((1, H), lambda i: (0, 0)),
            pl.BlockSpec((1, H), lambda i: (0, 0)),
            pl.BlockSpec((1, H), lambda i: (0, 0)),
        ],
        out_specs=pl.BlockSpec((BN, H), lambda i: (i, 0)),
        out_shape=jax.ShapeDtypeStruct((N, H), jnp.float32),
    )(h, a0, a1, Wgh, Wga, bg, g, b)


def _pool_head(h, batch3, Wp1, bp1, gp1, bp1_ln, Wp2, bp2, gp2, bp2_ln):
    nsteps = N // BN

    def body(h_ref, b_ref, Wp1_ref, bp1_ref, gp1_ref, bp1ln_ref,
             Wp2_ref, bp2_ref, gp2_ref, bp2ln_ref, zz_ref, xg_ref,
             s_acc, c_acc):
        i = pl.program_id(0)

        @pl.when(i == 0)
        def _():
            s_acc[...] = jnp.zeros_like(s_acc)
            c_acc[...] = jnp.zeros_like(c_acc)

        bvals = b_ref[0]
        onehotT = (lax.broadcasted_iota(jnp.int32, (G, BN), 0)
                   == jnp.broadcast_to(bvals, (G, BN))).astype(jnp.float32)
        s_acc[...] += jnp.dot(onehotT, h_ref[...], preferred_element_type=jnp.float32)
        c_acc[...] += jnp.broadcast_to(
            jnp.sum(onehotT, axis=1, keepdims=True), (G, H))

        @pl.when(i == nsteps - 1)
        def _():
            s = s_acc[...]
            cnt = c_acc[...][:, 0:1]
            x_mean = s / jnp.maximum(cnt, 1.0)
            x_add = s / (cnt + 1e-06)
            xg = x_mean + x_add
            z1 = jnp.dot(xg, Wp1_ref[...], preferred_element_type=jnp.float32)
            z1 = _silu(_ln(z1 + bp1_ref[...], gp1_ref[...], bp1ln_ref[...]))
            zz = jnp.dot(z1, Wp2_ref[...], preferred_element_type=jnp.float32)
            zz = _ln(zz + bp2_ref[...], gp2_ref[...], bp2ln_ref[...])
            zz_ref[...] = zz
            xg_ref[...] = xg

    return pl.pallas_call(
        body,
        grid=(nsteps,),
        in_specs=[
            pl.BlockSpec((BN, H), lambda i: (i, 0)),
            pl.BlockSpec((1, 1, BN), lambda i: (i, 0, 0)),
            pl.BlockSpec((H, H), lambda i: (0, 0)),
            pl.BlockSpec((1, H), lambda i: (0, 0)),
            pl.BlockSpec((1, H), lambda i: (0, 0)),
            pl.BlockSpec((1, H), lambda i: (0, 0)),
            pl.BlockSpec((H, DLAT), lambda i: (0, 0)),
            pl.BlockSpec((1, DLAT), lambda i: (0, 0)),
            pl.BlockSpec((1, DLAT), lambda i: (0, 0)),
            pl.BlockSpec((1, DLAT), lambda i: (0, 0)),
        ],
        out_specs=[
            pl.BlockSpec((G, DLAT), lambda i: (0, 0)),
            pl.BlockSpec((G, H), lambda i: (0, 0)),
        ],
        out_shape=[
            jax.ShapeDtypeStruct((G, DLAT), jnp.float32),
            jax.ShapeDtypeStruct((G, H), jnp.float32),
        ],
        scratch_shapes=[
            pltpu.VMEM((G, H), jnp.float32),
            pltpu.VMEM((G, H), jnp.float32),
        ],
    )(h, batch3, Wp1, bp1, gp1, bp1_ln, Wp2, bp2, gp2, bp2_ln)


def kernel(x, edge_index, edge_attr, batch, W_emb, b_emb, g_emb, be_emb,
           Wm1, bm1, Wm2, bm2, Wg, bg, g_ln, b_ln, Wp1, bp1, gp1, bp1_ln,
           Wp2, bp2, gp2, bp2_ln):
    f32 = jnp.float32
    src = edge_index[0]
    dst = edge_index[1]

    xp = jnp.pad(x, ((0, 0), (0, 8 - x.shape[1])))
    Wp = jnp.pad(W_emb, ((0, 8 - W_emb.shape[0]), (0, 0)))
    eap = jnp.pad(edge_attr, ((0, 0), (0, 8 - DE)))
    zeros_nh = jnp.zeros((N, H), f32)

    r = lambda v: v.reshape(1, -1)

    src3 = src[:EMAIN].reshape(NW, NFULL, EBLK)
    dst3 = dst[:EMAIN].reshape(NW, NFULL, EBLK)

    h = _embed(xp, Wp, r(b_emb), r(g_emb), r(be_emb))

    for l in range(L):
        hs, hd = _sc_gather(-1, h, src3, dst3, src, dst)
        bf16 = jnp.bfloat16
        W1d = Wm1[l, :H].astype(bf16)
        W1s = Wm1[l, H:2 * H].astype(bf16)
        W1e = jnp.pad(Wm1[l, 2 * H:], ((0, 8 - DE), (0, 0))).astype(bf16)
        m = _edge_mlp(hd, hs, eap.astype(bf16), W1d, W1s, W1e, r(bm1[l]),
                      Wm2[l].astype(bf16), r(bm2[l]))
        p = _sc_scatter(-1, m, dst3, dst, zeros_nh)
        h = _node_update(h, p[0], p[1],
                         Wg[l, :H], Wg[l, H:], r(bg[l]), r(g_ln[l]), r(b_ln[l]))

    batch3 = batch.reshape(N // BN, 1, BN)
    zz, xg = _pool_head(h, batch3, Wp1, r(bp1), r(gp1), r(bp1_ln),
                        Wp2, r(bp2), r(gp2), r(bp2_ln))
    return (zz, xg)


# speedup vs baseline: 1.0037x; 1.0037x over previous
import functools

import jax
import jax.numpy as jnp
from jax import lax
from jax.experimental import pallas as pl
from jax.experimental.pallas import tpu as pltpu
from jax.experimental.pallas import tpu_sc as plsc

N = 10000
E = 320000
H = 128
DE = 4
L = 3
DLAT = 64
G = 64

NW = 32
PW = E // NW
EBLK = 128
NFULL = PW // EBLK
TAIL = PW - NFULL * EBLK
EMAIN = NW * NFULL * EBLK
RPS = 624

CH = (30, 48)
E0 = NW * CH[0] * EBLK
E1M = NW * CH[1] * EBLK
E1P = E1M + 768

BE = 2000
BN = 2000

def _ln(x, g, b, eps=1e-05):
    mu = jnp.mean(x, axis=-1, keepdims=True)
    var = jnp.mean((x - mu) ** 2, axis=-1, keepdims=True)
    return (x - mu) / jnp.sqrt(var + eps) * g + b


def _silu(x):
    return x * jax.nn.sigmoid(x)


def _pack_bf16_pair(h):
    ua = lax.bitcast_convert_type(h[:, :H // 2], jnp.uint32)
    ub = lax.bitcast_convert_type(h[:, H // 2:], jnp.uint32)
    ua = (ua + jnp.uint32(0x8000)) >> 16
    ub = (ub + jnp.uint32(0x8000)) >> 16
    return lax.bitcast_convert_type(ua | (ub << 16), jnp.int32)


def _unpack_bf16_pair(w):
    u = lax.bitcast_convert_type(w, jnp.uint32)
    lo = lax.bitcast_convert_type(u << 16, jnp.float32)
    hi = lax.bitcast_convert_type(u & jnp.uint32(0xFFFF0000), jnp.float32)
    return lo.astype(jnp.bfloat16), hi.astype(jnp.bfloat16)


@functools.lru_cache(maxsize=None)
def _sc_gather_kernel(chunk):
    mesh = plsc.VectorSubcoreMesh(core_axis_name="c", subcore_axis_name="s")
    nb = NFULL if chunk == -1 else CH[chunk]
    eout = E if chunk == -1 else (E0 if chunk == 0 else E1P)
    tout0 = EMAIN if chunk == -1 else E1M

    @functools.partial(
        pl.kernel,
        out_type=(
            jax.ShapeDtypeStruct((eout, H), jnp.float32),
            jax.ShapeDtypeStruct((eout, H), jnp.float32),
        ),
        mesh=mesh,
        scratch_types=[
            pltpu.VMEM((nb, EBLK), jnp.int32),
            pltpu.VMEM((nb, EBLK), jnp.int32),
            pltpu.VMEM((EBLK, H), jnp.float32),
            pltpu.VMEM((EBLK, H), jnp.float32),
            pltpu.VMEM((EBLK, H), jnp.float32),
            pltpu.VMEM((EBLK, H), jnp.float32),
            pltpu.VMEM((TAIL,), jnp.int32),
            pltpu.VMEM((TAIL,), jnp.int32),
            pltpu.VMEM((TAIL, H), jnp.float32),
            pltpu.VMEM((TAIL, H), jnp.float32),
        ] + [pltpu.SemaphoreType.DMA] * 8,
    )
    def k(h_hbm, src3_hbm, dst3_hbm, src_hbm, dst_hbm, os_hbm, od_hbm,
          is2, id2, bs0, bd0, bs1, bd1, si_t, di_t, sr_t, dr_t,
          gs0, gd0, gs1, gd1, ws0, wd0, ws1, wd1):
        wid = lax.axis_index("c") * 16 + lax.axis_index("s")
        base = wid * (nb * EBLK)
        pltpu.sync_copy(src3_hbm.at[wid], is2)
        pltpu.sync_copy(dst3_hbm.at[wid], id2)

        @pl.loop(0, nb // 2)
        def _(p):
            b0 = 2 * p
            b1 = b0 + 1
            o0 = base + b0 * EBLK
            o1 = base + b1 * EBLK

            @pl.when(p > 0)
            def _():
                pltpu.make_async_copy(bs0, os_hbm.at[pl.ds(o0 - 2 * EBLK, EBLK)],
                                      ws0).wait()
                pltpu.make_async_copy(bd0, od_hbm.at[pl.ds(o0 - 2 * EBLK, EBLK)],
                                      wd0).wait()

            g0 = pltpu.async_copy(h_hbm.at[is2.at[b0]], bs0, gs0)
            g1 = pltpu.async_copy(h_hbm.at[id2.at[b0]], bd0, gd0)

            @pl.when(p > 0)
            def _():
                pltpu.make_async_copy(bs1, os_hbm.at[pl.ds(o1 - 2 * EBLK, EBLK)],
                                      ws1).wait()
                pltpu.make_async_copy(bd1, od_hbm.at[pl.ds(o1 - 2 * EBLK, EBLK)],
                                      wd1).wait()

            g2 = pltpu.async_copy(h_hbm.at[is2.at[b1]], bs1, gs1)
            g3 = pltpu.async_copy(h_hbm.at[id2.at[b1]], bd1, gd1)
            g0.wait()
            g1.wait()
            pltpu.async_copy(bs0, os_hbm.at[pl.ds(o0, EBLK)], ws0)
            pltpu.async_copy(bd0, od_hbm.at[pl.ds(o0, EBLK)], wd0)
            g2.wait()
            g3.wait()
            pltpu.async_copy(bs1, os_hbm.at[pl.ds(o1, EBLK)], ws1)
            pltpu.async_copy(bd1, od_hbm.at[pl.ds(o1, EBLK)], wd1)

        lo0 = base + (nb - 2) * EBLK
        lo1 = base + (nb - 1) * EBLK
        pltpu.make_async_copy(bs0, os_hbm.at[pl.ds(lo0, EBLK)], ws0).wait()
        pltpu.make_async_copy(bd0, od_hbm.at[pl.ds(lo0, EBLK)], wd0).wait()
        pltpu.make_async_copy(bs1, os_hbm.at[pl.ds(lo1, EBLK)], ws1).wait()
        pltpu.make_async_copy(bd1, od_hbm.at[pl.ds(lo1, EBLK)], wd1).wait()

        if chunk != 0:
            tin = EMAIN + wid * TAIL
            tout = tout0 + wid * TAIL
            pltpu.sync_copy(src_hbm.at[pl.ds(tin, TAIL)], si_t)
            pltpu.sync_copy(dst_hbm.at[pl.ds(tin, TAIL)], di_t)
            pltpu.sync_copy(h_hbm.at[si_t], sr_t)
            pltpu.sync_copy(h_hbm.at[di_t], dr_t)
            pltpu.sync_copy(sr_t, os_hbm.at[pl.ds(tout, TAIL)])
            pltpu.sync_copy(dr_t, od_hbm.at[pl.ds(tout, TAIL)])

    return k


def _sc_gather(chunk, h, src3, dst3, src, dst):
    return _sc_gather_kernel(chunk)(h, src3, dst3, src, dst)


@functools.lru_cache(maxsize=None)
def _sc_scatter_kernel(chunk):
    mesh = plsc.VectorSubcoreMesh(core_axis_name="c", subcore_axis_name="s")
    nb = NFULL if chunk == -1 else CH[chunk]
    ein = E if chunk == -1 else (E0 if chunk == 0 else E1P)
    tm0 = EMAIN if chunk == -1 else E1M

    @functools.partial(
        pl.kernel,
        out_type=jax.ShapeDtypeStruct((2, N, H), jnp.float32),
        mesh=mesh,
        scratch_types=[
            pltpu.VMEM((nb, EBLK), jnp.int32),
            pltpu.VMEM((EBLK, H), jnp.float32),
            pltpu.VMEM((EBLK, H), jnp.float32),
            pltpu.VMEM((TAIL,), jnp.int32),
            pltpu.VMEM((TAIL, H), jnp.float32),
            pltpu.VMEM_SHARED((N, H), jnp.float32),
        ] + [pltpu.SemaphoreType.DMA] * 2,
    )
    def k(m_hbm, dst3_hbm, dst_hbm, z_hbm, out_hbm,
          id2, ba, bb, di_t, mr_t, acc_sh, sa, sb):
        cid = lax.axis_index("c")
        sid = lax.axis_index("s")
        wid = cid * 16 + sid
        r0 = sid * RPS

        @pl.when(sid < 15)
        def _():
            pltpu.sync_copy(z_hbm.at[pl.ds(r0, RPS)], acc_sh.at[pl.ds(r0, RPS)])

        @pl.when(sid == 15)
        def _():
            pltpu.sync_copy(z_hbm.at[pl.ds(15 * RPS, N - 15 * RPS)],
                            acc_sh.at[pl.ds(15 * RPS, N - 15 * RPS)])

        pltpu.sync_copy(dst3_hbm.at[wid], id2)
        plsc.subcore_barrier()

        base = wid * (nb * EBLK)
        pltpu.async_copy(m_hbm.at[pl.ds(base, EBLK)], ba, sa).wait()

        @pl.loop(0, nb // 2)
        def _(p):
            b0 = 2 * p
            o1 = base + (b0 + 1) * EBLK
            lb = pltpu.async_copy(m_hbm.at[pl.ds(o1, EBLK)], bb, sb)
            pltpu.sync_copy(ba, acc_sh.at[id2.at[b0]], add=True)
            lb.wait()

            @pl.when(p < nb // 2 - 1)
            def _():
                o2 = base + (b0 + 2) * EBLK
                la = pltpu.async_copy(m_hbm.at[pl.ds(o2, EBLK)], ba, sa)
                pltpu.sync_copy(bb, acc_sh.at[id2.at[b0 + 1]], add=True)
                la.wait()

            @pl.when(p == nb // 2 - 1)
            def _():
                pltpu.sync_copy(bb, acc_sh.at[id2.at[b0 + 1]], add=True)

        if chunk != 0:
            tin = EMAIN + wid * TAIL
            tm = tm0 + wid * TAIL
            pltpu.sync_copy(dst_hbm.at[pl.ds(tin, TAIL)], di_t)
            pltpu.sync_copy(m_hbm.at[pl.ds(tm, TAIL)], mr_t)
            pltpu.sync_copy(mr_t, acc_sh.at[di_t], add=True)

        plsc.subcore_barrier()

        @pl.when(sid < 15)
        def _():
            pltpu.sync_copy(acc_sh.at[pl.ds(r0, RPS)],
                            out_hbm.at[cid, pl.ds(r0, RPS)])

        @pl.when(sid == 15)
        def _():
            pltpu.sync_copy(acc_sh.at[pl.ds(15 * RPS, N - 15 * RPS)],
                            out_hbm.at[cid, pl.ds(15 * RPS, N - 15 * RPS)])

    return k


def _sc_scatter(chunk, m, dst3, dst, zeros_nh):
    return _sc_scatter_kernel(chunk)(m, dst3, dst, zeros_nh)


def _embed(xp, Wp, b, g, be):
    def body(x_ref, W_ref, b_ref, g_ref, be_ref, o_ref):
        h = jnp.dot(x_ref[...], W_ref[...], preferred_element_type=jnp.float32)
        h = h + b_ref[...]
        o_ref[...] = _silu(_ln(h, g_ref[...], be_ref[...]))

    return pl.pallas_call(
        body,
        grid=(N // BN,),
        in_specs=[
            pl.BlockSpec((BN, 8), lambda i: (i, 0)),
            pl.BlockSpec((8, H), lambda i: (0, 0)),
            pl.BlockSpec((1, H), lambda i: (0, 0)),
            pl.BlockSpec((1, H), lambda i: (0, 0)),
            pl.BlockSpec((1, H), lambda i: (0, 0)),
        ],
        out_specs=pl.BlockSpec((BN, H), lambda i: (i, 0)),
        out_shape=jax.ShapeDtypeStruct((N, H), jnp.float32),
    )(xp, Wp, b, g, be)


def _edge_mlp(hd, hs, ea, W1d, W1s, W1e, b1, W2, b2):
    def body(hd_ref, hs_ref, ea_ref, W1d_ref, W1s_ref, W1e_ref, b1_ref,
             W2_ref, b2_ref, o_ref):
        bf16 = jnp.bfloat16
        z = jnp.dot(hd_ref[...].astype(bf16), W1d_ref[...],
                    preferred_element_type=jnp.float32)
        z = z + jnp.dot(hs_ref[...].astype(bf16), W1s_ref[...],
                        preferred_element_type=jnp.float32)
        z = z + jnp.dot(ea_ref[...], W1e_ref[...],
                        preferred_element_type=jnp.float32)
        z = _silu(z + b1_ref[...]).astype(bf16)
        o_ref[...] = jnp.dot(z, W2_ref[...], preferred_element_type=jnp.float32) + b2_ref[...]

    ne = hd.shape[0]
    return pl.pallas_call(
        body,
        grid=(ne // BE,),
        in_specs=[
            pl.BlockSpec((BE, H), lambda i: (i, 0)),
            pl.BlockSpec((BE, H), lambda i: (i, 0)),
            pl.BlockSpec((BE, 8), lambda i: (i, 0)),
            pl.BlockSpec((H, 2 * H), lambda i: (0, 0)),
            pl.BlockSpec((H, 2 * H), lambda i: (0, 0)),
            pl.BlockSpec((8, 2 * H), lambda i: (0, 0)),
            pl.BlockSpec((1, 2 * H), lambda i: (0, 0)),
            pl.BlockSpec((2 * H, H), lambda i: (0, 0)),
            pl.BlockSpec((1, H), lambda i: (0, 0)),
        ],
        out_specs=pl.BlockSpec((BE, H), lambda i: (i, 0)),
        out_shape=jax.ShapeDtypeStruct((ne, H), jnp.float32),
    )(hd, hs, ea, W1d, W1s, W1e, b1, W2, b2)


def _node_update(h, a0, a1, Wgh, Wga, bg, g, b):
    def body(h_ref, a0_ref, a1_ref, Wgh_ref, Wga_ref, bg_ref,
             g_ref, b_ref, o_ref):
        h = h_ref[...]
        a = a0_ref[...] + a1_ref[...]
        gate = jnp.dot(h, Wgh_ref[...], preferred_element_type=jnp.float32)
        gate = gate + jnp.dot(a, Wga_ref[...], preferred_element_type=jnp.float32)
        gate = jax.nn.sigmoid(gate + bg_ref[...])
        h_new = gate * a + (1.0 - gate) * h
        o_ref[...] = _ln(h + h_new, g_ref[...], b_ref[...])

    return pl.pallas_call(
        body,
        grid=(N // BN,),
        in_specs=[
            pl.BlockSpec((BN, H), lambda i: (i, 0)),
            pl.BlockSpec((BN, H), lambda i: (i, 0)),
            pl.BlockSpec((BN, H), lambda i: (i, 0)),
            pl.BlockSpec((H, H), lambda i: (0, 0)),
            pl.BlockSpec((H, H), lambda i: (0, 0)),
            pl.BlockSpec((1, H), lambda i: (0, 0)),
            pl.BlockSpec((1, H), lambda i: (0, 0)),
            pl.BlockSpec((1, H), lambda i: (0, 0)),
        ],
        out_specs=pl.BlockSpec((BN, H), lambda i: (i, 0)),
        out_shape=jax.ShapeDtypeStruct((N, H), jnp.float32),
    )(h, a0, a1, Wgh, Wga, bg, g, b)


def _pool_head(h, a0, a1, Wgh, Wga, bg, g, b,
               batch3, Wp1, bp1, gp1, bp1_ln, Wp2, bp2, gp2, bp2_ln):
    nsteps = N // BN

    def body(h_ref, a0_ref, a1_ref, Wgh_ref, Wga_ref, bg_ref, g_ref, b_ref,
             b3_ref, Wp1_ref, bp1_ref, gp1_ref, bp1ln_ref,
             Wp2_ref, bp2_ref, gp2_ref, bp2ln_ref, zz_ref, xg_ref,
             s_acc, c_acc):
        i = pl.program_id(0)

        @pl.when(i == 0)
        def _():
            s_acc[...] = jnp.zeros_like(s_acc)
            c_acc[...] = jnp.zeros_like(c_acc)

        hh = h_ref[...]
        a = a0_ref[...] + a1_ref[...]
        gate = jnp.dot(hh, Wgh_ref[...], preferred_element_type=jnp.float32)
        gate = gate + jnp.dot(a, Wga_ref[...], preferred_element_type=jnp.float32)
        gate = jax.nn.sigmoid(gate + bg_ref[...])
        h_new = gate * a + (1.0 - gate) * hh
        hn = _ln(hh + h_new, g_ref[...], b_ref[...])

        bvals = b3_ref[0]
        onehotT = (lax.broadcasted_iota(jnp.int32, (G, BN), 0)
                   == jnp.broadcast_to(bvals, (G, BN))).astype(jnp.float32)
        s_acc[...] += jnp.dot(onehotT, hn, preferred_element_type=jnp.float32)
        c_acc[...] += jnp.broadcast_to(
            jnp.sum(onehotT, axis=1, keepdims=True), (G, H))

        @pl.when(i == nsteps - 1)
        def _():
            s = s_acc[...]
            cnt = c_acc[...][:, 0:1]
            x_mean = s / jnp.maximum(cnt, 1.0)
            x_add = s / (cnt + 1e-06)
            xg = x_mean + x_add
            z1 = jnp.dot(xg, Wp1_ref[...], preferred_element_type=jnp.float32)
            z1 = _silu(_ln(z1 + bp1_ref[...], gp1_ref[...], bp1ln_ref[...]))
            zz = jnp.dot(z1, Wp2_ref[...], preferred_element_type=jnp.float32)
            zz = _ln(zz + bp2_ref[...], gp2_ref[...], bp2ln_ref[...])
            zz_ref[...] = zz
            xg_ref[...] = xg

    return pl.pallas_call(
        body,
        grid=(nsteps,),
        in_specs=[
            pl.BlockSpec((BN, H), lambda i: (i, 0)),
            pl.BlockSpec((BN, H), lambda i: (i, 0)),
            pl.BlockSpec((BN, H), lambda i: (i, 0)),
            pl.BlockSpec((H, H), lambda i: (0, 0)),
            pl.BlockSpec((H, H), lambda i: (0, 0)),
            pl.BlockSpec((1, H), lambda i: (0, 0)),
            pl.BlockSpec((1, H), lambda i: (0, 0)),
            pl.BlockSpec((1, H), lambda i: (0, 0)),
            pl.BlockSpec((1, 1, BN), lambda i: (i, 0, 0)),
            pl.BlockSpec((H, H), lambda i: (0, 0)),
            pl.BlockSpec((1, H), lambda i: (0, 0)),
            pl.BlockSpec((1, H), lambda i: (0, 0)),
            pl.BlockSpec((1, H), lambda i: (0, 0)),
            pl.BlockSpec((H, DLAT), lambda i: (0, 0)),
            pl.BlockSpec((1, DLAT), lambda i: (0, 0)),
            pl.BlockSpec((1, DLAT), lambda i: (0, 0)),
            pl.BlockSpec((1, DLAT), lambda i: (0, 0)),
        ],
        out_specs=[
            pl.BlockSpec((G, DLAT), lambda i: (0, 0)),
            pl.BlockSpec((G, H), lambda i: (0, 0)),
        ],
        out_shape=[
            jax.ShapeDtypeStruct((G, DLAT), jnp.float32),
            jax.ShapeDtypeStruct((G, H), jnp.float32),
        ],
        scratch_shapes=[
            pltpu.VMEM((G, H), jnp.float32),
            pltpu.VMEM((G, H), jnp.float32),
        ],
    )(h, a0, a1, Wgh, Wga, bg, g, b,
      batch3, Wp1, bp1, gp1, bp1_ln, Wp2, bp2, gp2, bp2_ln)


def kernel(x, edge_index, edge_attr, batch, W_emb, b_emb, g_emb, be_emb,
           Wm1, bm1, Wm2, bm2, Wg, bg, g_ln, b_ln, Wp1, bp1, gp1, bp1_ln,
           Wp2, bp2, gp2, bp2_ln):
    f32 = jnp.float32
    src = edge_index[0]
    dst = edge_index[1]

    xp = jnp.pad(x, ((0, 0), (0, 8 - x.shape[1])))
    Wp = jnp.pad(W_emb, ((0, 8 - W_emb.shape[0]), (0, 0)))
    eap = jnp.pad(edge_attr, ((0, 0), (0, 8 - DE)))
    zeros_nh = jnp.zeros((N, H), f32)

    r = lambda v: v.reshape(1, -1)

    src3 = src[:EMAIN].reshape(NW, NFULL, EBLK)
    dst3 = dst[:EMAIN].reshape(NW, NFULL, EBLK)

    h = _embed(xp, Wp, r(b_emb), r(g_emb), r(be_emb))

    bf16 = jnp.bfloat16
    eapb = eap.astype(bf16)
    p = None
    for l in range(L):
        hs, hd = _sc_gather(-1, h, src3, dst3, src, dst)
        W1d = Wm1[l, :H].astype(bf16)
        W1s = Wm1[l, H:2 * H].astype(bf16)
        W1e = jnp.pad(Wm1[l, 2 * H:], ((0, 8 - DE), (0, 0))).astype(bf16)
        m = _edge_mlp(hd, hs, eapb, W1d, W1s, W1e, r(bm1[l]),
                      Wm2[l].astype(bf16), r(bm2[l]))
        p = _sc_scatter(-1, m, dst3, dst, zeros_nh)
        if l < L - 1:
            h = _node_update(h, p[0], p[1], Wg[l, :H], Wg[l, H:],
                             r(bg[l]), r(g_ln[l]), r(b_ln[l]))

    batch3 = batch.reshape(N // BN, 1, BN)
    zz, xg = _pool_head(h, p[0], p[1], Wg[L - 1, :H], Wg[L - 1, H:],
                        r(bg[L - 1]), r(g_ln[L - 1]), r(b_ln[L - 1]),
                        batch3, Wp1, r(bp1), r(gp1), r(bp1_ln),
                        Wp2, r(bp2), r(gp2), r(bp2_ln))
    return (zz, xg)
